# Initial kernel scaffold; baseline (speedup 1.0000x reference)
import jax
import jax.numpy as jnp
from jax.experimental import pallas as pl


def kernel(xr, xi, edge_index, edge_attr, params):
    raise NotImplementedError("write your pallas kernel here")


# speedup vs baseline: 1.2573x; 1.2573x over previous
import functools

import jax
import jax.numpy as jnp
from jax import lax
from jax.experimental import pallas as pl
from jax.experimental.pallas import tpu as pltpu
from jax.experimental.pallas import tpu_sc as plsc

N_NODES = 10000
N_PAD = 10048
N_EDGES = 160000
E_PAD = 163840
NW = 32
EPW = E_PAD // NW
CHUNK = 128
NCHUNK = EPW // CHUNK
RPS = N_PAD // 16

_mesh = plsc.VectorSubcoreMesh(core_axis_name="c", subcore_axis_name="s")
_sc_params = pltpu.CompilerParams(use_tc_tiling_on_sc=False)


def _prelu(x, a):
    return jnp.where(x >= 0, x, a * x)


@functools.partial(
    pl.kernel,
    out_type=jax.ShapeDtypeStruct((E_PAD, 16), jnp.float32),
    mesh=_mesh,
    compiler_params=_sc_params,
    scratch_types=[
        pltpu.VMEM((NCHUNK, CHUNK), jnp.int32),
        pltpu.VMEM((CHUNK, 16), jnp.float32),
        pltpu.SemaphoreType.DMA,
    ],
)
def _gather_k(xcat_hbm, src_hbm, out_hbm, src_v, xs_v, sem):
    cid = lax.axis_index("c")
    sid = lax.axis_index("s")
    wid = sid * 2 + cid
    pltpu.sync_copy(src_hbm.at[wid], src_v)

    def body(c, _):
        pltpu.async_copy(xcat_hbm.at[src_v.at[c]], xs_v, sem).wait()
        pltpu.sync_copy(xs_v, out_hbm.at[pl.ds(wid * EPW + c * CHUNK, CHUNK)])
        return _

    lax.fori_loop(0, NCHUNK, body, 0)


@functools.partial(
    pl.kernel,
    out_type=jax.ShapeDtypeStruct((2, N_PAD, 32), jnp.float32),
    mesh=_mesh,
    compiler_params=_sc_params,
    scratch_types=[
        pltpu.VMEM((NCHUNK, CHUNK), jnp.int32),
        pltpu.VMEM((CHUNK, 32), jnp.float32),
        pltpu.VMEM_SHARED((N_PAD, 32), jnp.float32),
    ],
)
def _scatter_k(msg_hbm, dst_hbm, zeros_hbm, out_hbm, dst_v, msg_v, acc):
    cid = lax.axis_index("c")
    sid = lax.axis_index("s")
    wid = sid * 2 + cid
    pltpu.sync_copy(zeros_hbm.at[pl.ds(sid * RPS, RPS)],
                    acc.at[pl.ds(sid * RPS, RPS)])
    pltpu.sync_copy(dst_hbm.at[wid], dst_v)
    plsc.subcore_barrier()

    def body(c, _):
        pltpu.sync_copy(msg_hbm.at[pl.ds(wid * EPW + c * CHUNK, CHUNK)], msg_v)
        pltpu.sync_copy(msg_v, acc.at[dst_v.at[c]], add=True)
        return _

    lax.fori_loop(0, NCHUNK, body, 0)
    plsc.subcore_barrier()
    pltpu.sync_copy(acc.at[pl.ds(sid * RPS, RPS)],
                    out_hbm.at[cid, pl.ds(sid * RPS, RPS)])


@functools.partial(
    pl.kernel,
    out_type=jax.ShapeDtypeStruct((2, N_PAD, 8), jnp.float32),
    mesh=_mesh,
    compiler_params=_sc_params,
    scratch_types=[
        pltpu.VMEM((NCHUNK, CHUNK), jnp.int32),
        pltpu.VMEM((CHUNK, 8), jnp.float32),
        pltpu.VMEM_SHARED((N_PAD, 8), jnp.float32),
    ],
)
def _degree_k(dst_hbm, zeros_hbm, ones_hbm, out_hbm, dst_v, ones_v, acc):
    cid = lax.axis_index("c")
    sid = lax.axis_index("s")
    wid = sid * 2 + cid
    pltpu.sync_copy(zeros_hbm.at[pl.ds(sid * RPS, RPS)],
                    acc.at[pl.ds(sid * RPS, RPS)])
    pltpu.sync_copy(dst_hbm.at[wid], dst_v)
    pltpu.sync_copy(ones_hbm, ones_v)
    plsc.subcore_barrier()

    def body(c, _):
        pltpu.sync_copy(ones_v, acc.at[dst_v.at[c]], add=True)
        return _

    lax.fori_loop(0, NCHUNK, body, 0)
    plsc.subcore_barrier()
    pltpu.sync_copy(acc.at[pl.ds(sid * RPS, RPS)],
                    out_hbm.at[cid, pl.ds(sid * RPS, RPS)])


BE = 1024


def _msg_body(ea_ref, xs_ref,
              w1r, b1r, w2r, b2r, w3r, b3r, w4r, b4r,
              w1i, b1i, w2i, b2i, w3i, b3i, w4i, b4i,
              al, out_ref):
    ea = ea_ref[...]
    a1r, a2r, a3r = al[0, 0], al[0, 1], al[0, 2]
    a1i, a2i, a3i = al[0, 3], al[0, 4], al[0, 5]

    def mlp(w1, b1, w2, b2, w3, b3, w4, b4, a1, a2, a3):
        h = _prelu(jnp.dot(ea, w1[...], preferred_element_type=jnp.float32) + b1[...], a1)
        h = _prelu(jnp.dot(h, w2[...], preferred_element_type=jnp.float32) + b2[...], a2)
        h = _prelu(jnp.dot(h, w3[...], preferred_element_type=jnp.float32) + b3[...], a3)
        return jnp.dot(h, w4[...], preferred_element_type=jnp.float32) + b4[...]

    wer = mlp(w1r, b1r, w2r, b2r, w3r, b3r, w4r, b4r, a1r, a2r, a3r)
    wei = mlp(w1i, b1i, w2i, b2i, w3i, b3i, w4i, b4i, a1i, a2i, a3i)
    xs = xs_ref[...]
    xsr = xs[:, 0:8]
    xsi = xs[:, 8:16]

    def contract(x8, we):
        acc = x8[:, 0:1] * we[:, 0:8]
        for i in range(1, 8):
            acc = acc + x8[:, i:i + 1] * we[:, 8 * i:8 * i + 8]
        return acc

    out_ref[:, 0:8] = contract(xsr, wer)
    out_ref[:, 8:16] = contract(xsi, wer)
    out_ref[:, 16:24] = contract(xsr, wei)
    out_ref[:, 24:32] = contract(xsi, wei)


def _msg_call(ea_pad, xs, mp):
    full = lambda shape: pl.BlockSpec(shape, lambda e: (0,) * len(shape))
    wspecs = []
    wargs = []
    for nm in ("real_mlp", "imag_mlp"):
        m = mp[nm]
        for l in ("l1", "l2", "l3", "l4"):
            wargs += [m[l]["W"], m[l]["b"].reshape(1, -1)]
            wspecs += [full(m[l]["W"].shape), full((1, m[l]["b"].shape[0]))]
    al = jnp.stack([mp["real_mlp"]["a1"], mp["real_mlp"]["a2"], mp["real_mlp"]["a3"],
                    mp["imag_mlp"]["a1"], mp["imag_mlp"]["a2"], mp["imag_mlp"]["a3"],
                    jnp.float32(0.0), jnp.float32(0.0)]).reshape(1, 8)
    return pl.pallas_call(
        _msg_body,
        grid=(E_PAD // BE,),
        in_specs=[pl.BlockSpec((BE, 4), lambda e: (e, 0)),
                  pl.BlockSpec((BE, 16), lambda e: (e, 0))] + wspecs +
                 [pl.BlockSpec((1, 8), lambda e: (0, 0), memory_space=pltpu.SMEM)],
        out_specs=pl.BlockSpec((BE, 32), lambda e: (e, 0)),
        out_shape=jax.ShapeDtypeStruct((E_PAD, 32), jnp.float32),
    )(ea_pad, xs, *wargs, al)


def _combine_body(s_ref, deg_ref, x_ref, rootr, biasr, rooti, biasi, al, out_ref):
    ssum = s_ref[0] + s_ref[1]
    cnt = jnp.maximum(deg_ref[0] + deg_ref[1], 1.0)
    x = x_ref[...]
    xr = x[:, 0:8]
    xi = x[:, 8:16]
    rtr = jnp.dot(xr, rootr[...], preferred_element_type=jnp.float32) + biasr[...]
    rti = jnp.dot(xi, rootr[...], preferred_element_type=jnp.float32) + biasr[...]
    str_ = jnp.dot(xr, rooti[...], preferred_element_type=jnp.float32) + biasi[...]
    sti = jnp.dot(xi, rooti[...], preferred_element_type=jnp.float32) + biasi[...]
    rr = ssum[:, 0:8] / cnt + rtr
    ri = ssum[:, 8:16] / cnt + rti
    ir = ssum[:, 16:24] / cnt + str_
    ii = ssum[:, 24:32] / cnt + sti
    ar, ai = al[0, 0], al[0, 1]
    out_ref[:, 0:8] = _prelu(rr - ii, ar)
    out_ref[:, 8:16] = _prelu(ri + ir, ai)


NR = 1256


def _combine_call(s, deg, x, kp):
    al = jnp.stack([kp["ar"], kp["ai"]]).reshape(1, 2)
    full = lambda shape: pl.BlockSpec(shape, lambda r: (0,) * len(shape))
    return pl.pallas_call(
        _combine_body,
        grid=(N_PAD // NR,),
        in_specs=[pl.BlockSpec((2, NR, 32), lambda r: (0, r, 0)),
                  pl.BlockSpec((2, NR, 8), lambda r: (0, r, 0)),
                  pl.BlockSpec((NR, 16), lambda r: (r, 0)),
                  full((8, 8)), full((1, 8)), full((8, 8)), full((1, 8)),
                  pl.BlockSpec((1, 2), lambda r: (0, 0), memory_space=pltpu.SMEM)],
        out_specs=pl.BlockSpec((NR, 16), lambda r: (r, 0)),
        out_shape=jax.ShapeDtypeStruct((N_PAD, 16), jnp.float32),
    )(s, deg, x, kp["real_root"], kp["real_bias"].reshape(1, 8),
      kp["imag_root"], kp["imag_bias"].reshape(1, 8), al)


def _fcup_body(xr_ref, xi_ref, w1r, b1r, w2r, b2r, w1i, b1i, w2i, b2i, al, out_ref):
    hr = _prelu(jnp.dot(xr_ref[...], w1r[...], preferred_element_type=jnp.float32) + b1r[...], al[0, 0])
    hr = _prelu(jnp.dot(hr, w2r[...], preferred_element_type=jnp.float32) + b2r[...], al[0, 1])
    hi = _prelu(jnp.dot(xi_ref[...], w1i[...], preferred_element_type=jnp.float32) + b1i[...], al[0, 2])
    hi = _prelu(jnp.dot(hi, w2i[...], preferred_element_type=jnp.float32) + b2i[...], al[0, 3])
    out_ref[:, 0:8] = hr
    out_ref[:, 8:16] = hi


def _fcup_call(xr_pad, xi_pad, p):
    al = jnp.stack([p["aup1r"], p["aup2r"], p["aup1i"], p["aup2i"]]).reshape(1, 4)
    full = lambda shape: pl.BlockSpec(shape, lambda r: (0,) * len(shape))
    return pl.pallas_call(
        _fcup_body,
        grid=(N_PAD // NR,),
        in_specs=[pl.BlockSpec((NR, 1), lambda r: (r, 0)),
                  pl.BlockSpec((NR, 1), lambda r: (r, 0)),
                  full((1, 4)), full((1, 4)), full((4, 8)), full((1, 8)),
                  full((1, 4)), full((1, 4)), full((4, 8)), full((1, 8)),
                  pl.BlockSpec((1, 4), lambda r: (0, 0), memory_space=pltpu.SMEM)],
        out_specs=pl.BlockSpec((NR, 16), lambda r: (r, 0)),
        out_shape=jax.ShapeDtypeStruct((N_PAD, 16), jnp.float32),
    )(xr_pad, xi_pad,
      p["fcup1r"]["W"], p["fcup1r"]["b"].reshape(1, 4),
      p["fcup2r"]["W"], p["fcup2r"]["b"].reshape(1, 8),
      p["fcup1i"]["W"], p["fcup1i"]["b"].reshape(1, 4),
      p["fcup2i"]["W"], p["fcup2i"]["b"].reshape(1, 8), al)


def _fcdown_body(x_ref, w1r, b1r, w2r, b2r, w1i, b1i, w2i, b2i, al, hr_ref, hi_ref):
    x = x_ref[...]
    hr = _prelu(jnp.dot(x[:, 0:8], w1r[...], preferred_element_type=jnp.float32) + b1r[...], al[0, 0])
    hr_ref[...] = _prelu(jnp.dot(hr, w2r[...], preferred_element_type=jnp.float32) + b2r[...], al[0, 1])
    hi = _prelu(jnp.dot(x[:, 8:16], w1i[...], preferred_element_type=jnp.float32) + b1i[...], al[0, 2])
    hi_ref[...] = _prelu(jnp.dot(hi, w2i[...], preferred_element_type=jnp.float32) + b2i[...], al[0, 3])


def _fcdown_call(x, p):
    al = jnp.stack([p["adown1r"], p["adown2r"], p["adown1i"], p["adown2i"]]).reshape(1, 4)
    full = lambda shape: pl.BlockSpec(shape, lambda r: (0,) * len(shape))
    return pl.pallas_call(
        _fcdown_body,
        grid=(N_PAD // NR,),
        in_specs=[pl.BlockSpec((NR, 16), lambda r: (r, 0)),
                  full((8, 4)), full((1, 4)), full((4, 1)), full((1, 1)),
                  full((8, 4)), full((1, 4)), full((4, 1)), full((1, 1)),
                  pl.BlockSpec((1, 4), lambda r: (0, 0), memory_space=pltpu.SMEM)],
        out_specs=[pl.BlockSpec((NR, 1), lambda r: (r, 0)),
                   pl.BlockSpec((NR, 1), lambda r: (r, 0))],
        out_shape=[jax.ShapeDtypeStruct((N_PAD, 1), jnp.float32),
                   jax.ShapeDtypeStruct((N_PAD, 1), jnp.float32)],
    )(x, p["fcdown1r"]["W"], p["fcdown1r"]["b"].reshape(1, 4),
      p["fcdown2r"]["W"], p["fcdown2r"]["b"].reshape(1, 1),
      p["fcdown1i"]["W"], p["fcdown1i"]["b"].reshape(1, 4),
      p["fcdown2i"]["W"], p["fcdown2i"]["b"].reshape(1, 1), al)


def kernel(xr, xi, edge_index, edge_attr, params):
    src = edge_index[0]
    dst = edge_index[1]
    pad = E_PAD - N_EDGES
    src_r = jnp.concatenate([src, jnp.zeros((pad,), jnp.int32)]).reshape(NW, NCHUNK, CHUNK)
    dst_r = jnp.concatenate([dst, jnp.full((pad,), N_NODES + 8, jnp.int32)]).reshape(NW, NCHUNK, CHUNK)
    ea_pad = jnp.concatenate([edge_attr, jnp.zeros((pad, 4), jnp.float32)], axis=0)
    xr_pad = jnp.concatenate([xr, jnp.zeros((N_PAD - N_NODES, 1), jnp.float32)], axis=0)
    xi_pad = jnp.concatenate([xi, jnp.zeros((N_PAD - N_NODES, 1), jnp.float32)], axis=0)
    zeros32 = jnp.zeros((N_PAD, 32), jnp.float32)
    ones8 = jnp.ones((CHUNK, 8), jnp.float32)

    deg = _degree_k(dst_r, zeros32[:, :8], ones8)
    x = _fcup_call(xr_pad, xi_pad, params)
    for kp in params["kernels"]:
        xs = _gather_k(x, src_r)
        msg = _msg_call(ea_pad, xs, kp)
        s = _scatter_k(msg, dst_r, zeros32)
        x = _combine_call(s, deg, x, kp)
    hr, hi = _fcdown_call(x, params)
    return hr[:N_NODES], hi[:N_NODES]


# speedup vs baseline: 5.9732x; 4.7510x over previous
import functools

import jax
import jax.numpy as jnp
from jax import lax
from jax.experimental import pallas as pl
from jax.experimental.pallas import tpu as pltpu
from jax.experimental.pallas import tpu_sc as plsc

N_NODES = 10000
N_PAD = 10048
N_EDGES = 160000
E_PAD = 163840
NW = 32
EPW = E_PAD // NW
CHUNK = 128
NCHUNK = EPW // CHUNK
RPS = N_PAD // 16

_mesh = plsc.VectorSubcoreMesh(core_axis_name="c", subcore_axis_name="s")
_sc_params = pltpu.CompilerParams(use_tc_tiling_on_sc=False)


def _prelu(x, a):
    return jnp.where(x >= 0, x, a * x)


@functools.partial(
    pl.kernel,
    out_type=jax.ShapeDtypeStruct((E_PAD, 16), jnp.float32),
    mesh=_mesh,
    compiler_params=_sc_params,
    scratch_types=[
        pltpu.VMEM((NCHUNK, CHUNK), jnp.int32),
        pltpu.VMEM((EPW, 16), jnp.float32),
        pltpu.SemaphoreType.DMA,
    ],
)
def _gather_k(xcat_hbm, src_hbm, out_hbm, src_v, xs_v, sem):
    cid = lax.axis_index("c")
    sid = lax.axis_index("s")
    wid = sid * 2 + cid
    pltpu.sync_copy(src_hbm.at[wid], src_v)
    K = 10

    def body(t, _):
        handles = [
            pltpu.async_copy(xcat_hbm.at[src_v.at[t * K + j]],
                             xs_v.at[pl.ds((t * K + j) * CHUNK, CHUNK)], sem)
            for j in range(K)
        ]
        for h in handles:
            h.wait()
        return _

    lax.fori_loop(0, NCHUNK // K, body, 0)
    pltpu.sync_copy(xs_v, out_hbm.at[pl.ds(wid * EPW, EPW)])


@functools.partial(
    pl.kernel,
    out_type=jax.ShapeDtypeStruct((2, N_PAD, 32), jnp.float32),
    mesh=_mesh,
    compiler_params=_sc_params,
    scratch_types=[
        pltpu.VMEM((NCHUNK, CHUNK), jnp.int32),
        pltpu.VMEM((EPW // 2, 32), jnp.float32),
        pltpu.VMEM_SHARED((N_PAD, 32), jnp.float32),
        pltpu.SemaphoreType.DMA,
    ],
)
def _scatter_k(msg_hbm, dst_hbm, zeros_hbm, out_hbm, dst_v, msg_v, acc, sem):
    cid = lax.axis_index("c")
    sid = lax.axis_index("s")
    wid = sid * 2 + cid
    pltpu.sync_copy(zeros_hbm.at[pl.ds(sid * RPS, RPS)],
                    acc.at[pl.ds(sid * RPS, RPS)])
    pltpu.sync_copy(dst_hbm.at[wid], dst_v)
    plsc.subcore_barrier()
    K = 10
    HC = NCHUNK // 2

    def round_(r, _):
        pltpu.sync_copy(msg_hbm.at[pl.ds(wid * EPW + r * (EPW // 2), EPW // 2)],
                        msg_v)

        def body(t, _2):
            handles = [
                pltpu.async_copy(msg_v.at[pl.ds((t * K + j) * CHUNK, CHUNK)],
                                 acc.at[dst_v.at[r * HC + t * K + j]],
                                 sem, add=True)
                for j in range(K)
            ]
            for h in handles:
                h.wait()
            return _2

        lax.fori_loop(0, HC // K, body, 0)
        return _

    lax.fori_loop(0, 2, round_, 0)
    plsc.subcore_barrier()
    pltpu.sync_copy(acc.at[pl.ds(sid * RPS, RPS)],
                    out_hbm.at[cid, pl.ds(sid * RPS, RPS)])


@functools.partial(
    pl.kernel,
    out_type=jax.ShapeDtypeStruct((2, N_PAD, 8), jnp.float32),
    mesh=_mesh,
    compiler_params=_sc_params,
    scratch_types=[
        pltpu.VMEM((NCHUNK, CHUNK), jnp.int32),
        pltpu.VMEM((CHUNK, 8), jnp.float32),
        pltpu.VMEM_SHARED((N_PAD, 8), jnp.float32),
        pltpu.SemaphoreType.DMA,
    ],
)
def _degree_k(dst_hbm, zeros_hbm, ones_hbm, out_hbm, dst_v, ones_v, acc, sem):
    cid = lax.axis_index("c")
    sid = lax.axis_index("s")
    wid = sid * 2 + cid
    pltpu.sync_copy(zeros_hbm.at[pl.ds(sid * RPS, RPS)],
                    acc.at[pl.ds(sid * RPS, RPS)])
    pltpu.sync_copy(dst_hbm.at[wid], dst_v)
    pltpu.sync_copy(ones_hbm, ones_v)
    plsc.subcore_barrier()
    K = 4

    def body(t, _):
        handles = [
            pltpu.async_copy(ones_v, acc.at[dst_v.at[t * K + j]], sem, add=True)
            for j in range(K)
        ]
        for h in handles:
            h.wait()
        return _

    lax.fori_loop(0, NCHUNK // K, body, 0)
    plsc.subcore_barrier()
    pltpu.sync_copy(acc.at[pl.ds(sid * RPS, RPS)],
                    out_hbm.at[cid, pl.ds(sid * RPS, RPS)])


BE = 8192


def _msg_body(ea_ref, xs_ref,
              w1r, b1r, w2r, b2r, w3r, b3r, w4r, b4r,
              w1i, b1i, w2i, b2i, w3i, b3i, w4i, b4i,
              al, out_ref):
    ea = ea_ref[...]
    a1r, a2r, a3r = al[0, 0], al[0, 1], al[0, 2]
    a1i, a2i, a3i = al[0, 3], al[0, 4], al[0, 5]

    def mlp(w1, b1, w2, b2, w3, b3, w4, b4, a1, a2, a3):
        h = _prelu(jnp.dot(w1[...], ea, preferred_element_type=jnp.float32) + b1[...], a1)
        h = _prelu(jnp.dot(w2[...], h, preferred_element_type=jnp.float32) + b2[...], a2)
        h = _prelu(jnp.dot(w3[...], h, preferred_element_type=jnp.float32) + b3[...], a3)
        return jnp.dot(w4[...], h, preferred_element_type=jnp.float32) + b4[...]

    wer = mlp(w1r, b1r, w2r, b2r, w3r, b3r, w4r, b4r, a1r, a2r, a3r)
    wei = mlp(w1i, b1i, w2i, b2i, w3i, b3i, w4i, b4i, a1i, a2i, a3i)
    xs_t = xs_ref[...].T
    xsr = xs_t[0:8]
    xsi = xs_t[8:16]

    def contract(x8, we):
        acc = x8[0:1] * we[0:8]
        for i in range(1, 8):
            acc = acc + x8[i:i + 1] * we[8 * i:8 * i + 8]
        return acc

    msg = jnp.concatenate(
        [contract(xsr, wer), contract(xsi, wer),
         contract(xsr, wei), contract(xsi, wei)], axis=0)
    out_ref[...] = msg.T


def _msg_call(ea8t, xs, mp):
    full = lambda shape: pl.BlockSpec(shape, lambda e: (0,) * len(shape))
    wspecs = []
    wargs = []
    for nm in ("real_mlp", "imag_mlp"):
        m = mp[nm]
        for li, l in enumerate(("l1", "l2", "l3", "l4")):
            wt = m[l]["W"].T
            if l == "l1":
                wt = jnp.concatenate([wt, jnp.zeros((32, 4), jnp.float32)], axis=1)
            wargs += [wt, m[l]["b"].reshape(-1, 1)]
            wspecs += [full(wt.shape), full((m[l]["b"].shape[0], 1))]
    al = jnp.stack([mp["real_mlp"]["a1"], mp["real_mlp"]["a2"], mp["real_mlp"]["a3"],
                    mp["imag_mlp"]["a1"], mp["imag_mlp"]["a2"], mp["imag_mlp"]["a3"],
                    jnp.float32(0.0), jnp.float32(0.0)]).reshape(1, 8)
    return pl.pallas_call(
        _msg_body,
        grid=(E_PAD // BE,),
        in_specs=[pl.BlockSpec((8, BE), lambda e: (0, e)),
                  pl.BlockSpec((BE, 16), lambda e: (e, 0))] + wspecs +
                 [pl.BlockSpec((1, 8), lambda e: (0, 0), memory_space=pltpu.SMEM)],
        out_specs=pl.BlockSpec((BE, 32), lambda e: (e, 0)),
        out_shape=jax.ShapeDtypeStruct((E_PAD, 32), jnp.float32),
    )(ea8t, xs, *wargs, al)


def _combine_body(s_ref, cntinv_ref, x_ref, rootr, biasr, rooti, biasi, al, out_ref):
    ssum = s_ref[0] + s_ref[1]
    cinv = cntinv_ref[...]
    x = x_ref[...]
    xr = x[:, 0:8]
    xi = x[:, 8:16]
    rtr = jnp.dot(xr, rootr[...], preferred_element_type=jnp.float32) + biasr[...]
    rti = jnp.dot(xi, rootr[...], preferred_element_type=jnp.float32) + biasr[...]
    str_ = jnp.dot(xr, rooti[...], preferred_element_type=jnp.float32) + biasi[...]
    sti = jnp.dot(xi, rooti[...], preferred_element_type=jnp.float32) + biasi[...]
    rr = ssum[:, 0:8] * cinv + rtr
    ri = ssum[:, 8:16] * cinv + rti
    ir = ssum[:, 16:24] * cinv + str_
    ii = ssum[:, 24:32] * cinv + sti
    ar, ai = al[0, 0], al[0, 1]
    out_ref[:, 0:8] = _prelu(rr - ii, ar)
    out_ref[:, 8:16] = _prelu(ri + ir, ai)


NR = 1256


def _combine_call(s, cntinv, x, kp):
    al = jnp.stack([kp["ar"], kp["ai"]]).reshape(1, 2)
    full = lambda shape: pl.BlockSpec(shape, lambda r: (0,) * len(shape))
    return pl.pallas_call(
        _combine_body,
        grid=(N_PAD // NR,),
        in_specs=[pl.BlockSpec((2, NR, 32), lambda r: (0, r, 0)),
                  pl.BlockSpec((NR, 8), lambda r: (r, 0)),
                  pl.BlockSpec((NR, 16), lambda r: (r, 0)),
                  full((8, 8)), full((1, 8)), full((8, 8)), full((1, 8)),
                  pl.BlockSpec((1, 2), lambda r: (0, 0), memory_space=pltpu.SMEM)],
        out_specs=pl.BlockSpec((NR, 16), lambda r: (r, 0)),
        out_shape=jax.ShapeDtypeStruct((N_PAD, 16), jnp.float32),
    )(s, cntinv, x, kp["real_root"], kp["real_bias"].reshape(1, 8),
      kp["imag_root"], kp["imag_bias"].reshape(1, 8), al)


def _fcup_body(xr_ref, xi_ref, w1r, b1r, w2r, b2r, w1i, b1i, w2i, b2i, al, out_ref):
    hr = _prelu(jnp.dot(xr_ref[...], w1r[...], preferred_element_type=jnp.float32) + b1r[...], al[0, 0])
    hr = _prelu(jnp.dot(hr, w2r[...], preferred_element_type=jnp.float32) + b2r[...], al[0, 1])
    hi = _prelu(jnp.dot(xi_ref[...], w1i[...], preferred_element_type=jnp.float32) + b1i[...], al[0, 2])
    hi = _prelu(jnp.dot(hi, w2i[...], preferred_element_type=jnp.float32) + b2i[...], al[0, 3])
    out_ref[:, 0:8] = hr
    out_ref[:, 8:16] = hi


def _fcup_call(xr_pad, xi_pad, p):
    al = jnp.stack([p["aup1r"], p["aup2r"], p["aup1i"], p["aup2i"]]).reshape(1, 4)
    full = lambda shape: pl.BlockSpec(shape, lambda r: (0,) * len(shape))
    return pl.pallas_call(
        _fcup_body,
        grid=(N_PAD // NR,),
        in_specs=[pl.BlockSpec((NR, 1), lambda r: (r, 0)),
                  pl.BlockSpec((NR, 1), lambda r: (r, 0)),
                  full((1, 4)), full((1, 4)), full((4, 8)), full((1, 8)),
                  full((1, 4)), full((1, 4)), full((4, 8)), full((1, 8)),
                  pl.BlockSpec((1, 4), lambda r: (0, 0), memory_space=pltpu.SMEM)],
        out_specs=pl.BlockSpec((NR, 16), lambda r: (r, 0)),
        out_shape=jax.ShapeDtypeStruct((N_PAD, 16), jnp.float32),
    )(xr_pad, xi_pad,
      p["fcup1r"]["W"], p["fcup1r"]["b"].reshape(1, 4),
      p["fcup2r"]["W"], p["fcup2r"]["b"].reshape(1, 8),
      p["fcup1i"]["W"], p["fcup1i"]["b"].reshape(1, 4),
      p["fcup2i"]["W"], p["fcup2i"]["b"].reshape(1, 8), al)


def _fcdown_body(x_ref, w1r, b1r, w2r, b2r, w1i, b1i, w2i, b2i, al, hr_ref, hi_ref):
    x = x_ref[...]
    hr = _prelu(jnp.dot(x[:, 0:8], w1r[...], preferred_element_type=jnp.float32) + b1r[...], al[0, 0])
    hr_ref[...] = _prelu(jnp.dot(hr, w2r[...], preferred_element_type=jnp.float32) + b2r[...], al[0, 1])
    hi = _prelu(jnp.dot(x[:, 8:16], w1i[...], preferred_element_type=jnp.float32) + b1i[...], al[0, 2])
    hi_ref[...] = _prelu(jnp.dot(hi, w2i[...], preferred_element_type=jnp.float32) + b2i[...], al[0, 3])


def _fcdown_call(x, p):
    al = jnp.stack([p["adown1r"], p["adown2r"], p["adown1i"], p["adown2i"]]).reshape(1, 4)
    full = lambda shape: pl.BlockSpec(shape, lambda r: (0,) * len(shape))
    return pl.pallas_call(
        _fcdown_body,
        grid=(N_PAD // NR,),
        in_specs=[pl.BlockSpec((NR, 16), lambda r: (r, 0)),
                  full((8, 4)), full((1, 4)), full((4, 1)), full((1, 1)),
                  full((8, 4)), full((1, 4)), full((4, 1)), full((1, 1)),
                  pl.BlockSpec((1, 4), lambda r: (0, 0), memory_space=pltpu.SMEM)],
        out_specs=[pl.BlockSpec((NR, 1), lambda r: (r, 0)),
                   pl.BlockSpec((NR, 1), lambda r: (r, 0))],
        out_shape=[jax.ShapeDtypeStruct((N_PAD, 1), jnp.float32),
                   jax.ShapeDtypeStruct((N_PAD, 1), jnp.float32)],
    )(x, p["fcdown1r"]["W"], p["fcdown1r"]["b"].reshape(1, 4),
      p["fcdown2r"]["W"], p["fcdown2r"]["b"].reshape(1, 1),
      p["fcdown1i"]["W"], p["fcdown1i"]["b"].reshape(1, 4),
      p["fcdown2i"]["W"], p["fcdown2i"]["b"].reshape(1, 1), al)


def kernel(xr, xi, edge_index, edge_attr, params):
    src = edge_index[0]
    dst = edge_index[1]
    pad = E_PAD - N_EDGES
    src_r = jnp.concatenate([src, jnp.zeros((pad,), jnp.int32)]).reshape(NW, NCHUNK, CHUNK)
    dst_r = jnp.concatenate([dst, jnp.full((pad,), N_NODES + 8, jnp.int32)]).reshape(NW, NCHUNK, CHUNK)
    ea8t = jnp.concatenate(
        [edge_attr.T, jnp.zeros((pad, 4), jnp.float32).T], axis=1)
    ea8t = jnp.concatenate([ea8t, jnp.zeros((4, E_PAD), jnp.float32)], axis=0)
    xr_pad = jnp.concatenate([xr, jnp.zeros((N_PAD - N_NODES, 1), jnp.float32)], axis=0)
    xi_pad = jnp.concatenate([xi, jnp.zeros((N_PAD - N_NODES, 1), jnp.float32)], axis=0)
    zeros32 = jnp.zeros((N_PAD, 32), jnp.float32)
    ones8 = jnp.ones((CHUNK, 8), jnp.float32)

    deg = _degree_k(dst_r, zeros32[:, :8], ones8)
    cntinv = 1.0 / jnp.maximum(deg[0] + deg[1], 1.0)
    x = _fcup_call(xr_pad, xi_pad, params)
    for kp in params["kernels"]:
        xs = _gather_k(x, src_r)
        msg = _msg_call(ea8t, xs, kp)
        s = _scatter_k(msg, dst_r, zeros32)
        x = _combine_call(s, cntinv, x, kp)
    hr, hi = _fcdown_call(x, params)
    return hr[:N_NODES], hi[:N_NODES]
